# baseline (speedup 1.0000x reference)
import functools

import jax
import jax.numpy as jnp
import numpy as np
from jax import lax
from jax.experimental import pallas as pl
from jax.experimental.pallas import tpu as pltpu
from jax.experimental.pallas import tpu_sc as plsc

_NUM_PATCHES = 576
_NUM_MASK = 432
_NUM_KEEP = _NUM_PATCHES - _NUM_MASK
_BATCH = 64
_DIM = 768

_NC, _NS = 2, 16
_NW = _NC * _NS
_ROWS = _BATCH * _NUM_KEEP
_ROWS_PER_W = _ROWS // _NW
_CHUNKS = 4
_CHUNK = _ROWS_PER_W // _CHUNKS
_NBUF = 2


_KEEP = np.array([
    7, 10, 11, 12, 15, 16, 20, 23, 24, 25, 28, 29, 38, 44, 47, 55, 60, 61,
    68, 76, 82, 84, 87, 88, 93, 96, 111, 112, 113, 114, 119, 122, 128, 129,
    131, 135, 145, 148, 151, 152, 153, 154, 157, 168, 175, 178, 187, 188,
    199, 201, 202, 203, 209, 210, 212, 215, 217, 219, 222, 224, 225, 229,
    233, 235, 237, 238, 239, 240, 241, 245, 247, 248, 251, 255, 257, 259,
    262, 271, 278, 283, 284, 289, 290, 292, 299, 308, 313, 317, 321, 326,
    327, 332, 333, 334, 335, 339, 345, 346, 347, 356, 367, 369, 374, 382,
    383, 389, 390, 391, 393, 397, 400, 403, 413, 416, 420, 428, 432, 434,
    436, 439, 442, 444, 446, 448, 451, 454, 461, 472, 474, 478, 486, 489,
    492, 493, 495, 504, 507, 523, 528, 550, 555, 567, 569, 573,
], dtype=np.int32)
_FLAT_IDX = (
    (np.arange(_BATCH)[:, None] * _NUM_PATCHES + _KEEP[None, :])
    .reshape(_NW, _CHUNKS, _CHUNK)
    .astype(np.int32)
)


@functools.lru_cache(maxsize=1)
def _flat_indices():
    return jnp.asarray(_FLAT_IDX)


def _gather_body(table, idxs, out, idx_v, *rest):
    bufs = rest[:_NBUF]
    gsems = rest[_NBUF : 2 * _NBUF]
    wsems = rest[2 * _NBUF :]
    wid = lax.axis_index("s") * _NC + lax.axis_index("c")
    pltpu.sync_copy(idxs.at[wid], idx_v)

    def gather(g):
        return pltpu.async_copy(table.at[idx_v.at[g]], bufs[g % _NBUF], gsems[g % _NBUF])

    per_batch = _CHUNKS // 2

    def write(g):
        b = 2 * wid + g // per_batch
        r = (g % per_batch) * _CHUNK
        return pltpu.async_copy(
            bufs[g % _NBUF], out.at[b, pl.ds(r, _CHUNK)], wsems[g % _NBUF]
        )

    gc = [None] * _CHUNKS
    wc = [None] * _CHUNKS
    for g in range(_NBUF):
        gc[g] = gather(g)
    for g in range(_CHUNKS):
        gc[g].wait()
        wc[g] = write(g)
        if g + _NBUF < _CHUNKS:
            wc[g].wait()
            gc[g + _NBUF] = gather(g + _NBUF)
    for g in range(_CHUNKS - _NBUF, _CHUNKS):
        wc[g].wait()


def _run(table):
    mesh = plsc.VectorSubcoreMesh(
        core_axis_name="c", subcore_axis_name="s", num_cores=_NC, num_subcores=_NS
    )
    k = pl.kernel(
        _gather_body,
        out_type=jax.ShapeDtypeStruct((_BATCH, _NUM_KEEP, _DIM), jnp.float32),
        mesh=mesh,
        scratch_types=(
            [pltpu.VMEM((_CHUNKS, _CHUNK), jnp.int32)]
            + [pltpu.VMEM((_CHUNK, _DIM), jnp.float32)] * _NBUF
            + [pltpu.SemaphoreType.DMA] * (2 * _NBUF)
        ),
    )
    return k(table, _flat_indices())


def kernel(patches):
    table = patches.reshape(_BATCH * _NUM_PATCHES, _DIM)
    return _run(table)


# speedup vs baseline: 1.0173x; 1.0173x over previous
import functools

import jax
import jax.numpy as jnp
import numpy as np
from jax import lax
from jax.experimental import pallas as pl
from jax.experimental.pallas import tpu as pltpu
from jax.experimental.pallas import tpu_sc as plsc

_NUM_PATCHES = 576
_NUM_MASK = 432
_NUM_KEEP = _NUM_PATCHES - _NUM_MASK
_BATCH = 64
_DIM = 768

_NC, _NS = 2, 16
_NW = _NC * _NS
_ROWS = _BATCH * _NUM_KEEP
_ROWS_PER_W = _ROWS // _NW
_CHUNKS = 12
_CHUNK = _ROWS_PER_W // _CHUNKS
_NBUF = 6


_KEEP = np.array([
    7, 10, 11, 12, 15, 16, 20, 23, 24, 25, 28, 29, 38, 44, 47, 55, 60, 61,
    68, 76, 82, 84, 87, 88, 93, 96, 111, 112, 113, 114, 119, 122, 128, 129,
    131, 135, 145, 148, 151, 152, 153, 154, 157, 168, 175, 178, 187, 188,
    199, 201, 202, 203, 209, 210, 212, 215, 217, 219, 222, 224, 225, 229,
    233, 235, 237, 238, 239, 240, 241, 245, 247, 248, 251, 255, 257, 259,
    262, 271, 278, 283, 284, 289, 290, 292, 299, 308, 313, 317, 321, 326,
    327, 332, 333, 334, 335, 339, 345, 346, 347, 356, 367, 369, 374, 382,
    383, 389, 390, 391, 393, 397, 400, 403, 413, 416, 420, 428, 432, 434,
    436, 439, 442, 444, 446, 448, 451, 454, 461, 472, 474, 478, 486, 489,
    492, 493, 495, 504, 507, 523, 528, 550, 555, 567, 569, 573,
], dtype=np.int32)
_FLAT_IDX = (
    (np.arange(_BATCH)[:, None] * _NUM_PATCHES + _KEEP[None, :])
    .reshape(_NW, _CHUNKS, _CHUNK)
    .astype(np.int32)
)


@functools.lru_cache(maxsize=1)
def _flat_indices():
    return jnp.asarray(_FLAT_IDX)


def _gather_body(table, idxs, out, idx_v, *rest):
    bufs = rest[:_NBUF]
    gsems = rest[_NBUF : 2 * _NBUF]
    wsems = rest[2 * _NBUF :]
    wid = lax.axis_index("s") * _NC + lax.axis_index("c")
    pltpu.sync_copy(idxs.at[wid], idx_v)

    def gather(g):
        return pltpu.async_copy(table.at[idx_v.at[g]], bufs[g % _NBUF], gsems[g % _NBUF])

    per_batch = _CHUNKS // 2

    def write(g):
        b = 2 * wid + g // per_batch
        r = (g % per_batch) * _CHUNK
        return pltpu.async_copy(
            bufs[g % _NBUF], out.at[b, pl.ds(r, _CHUNK)], wsems[g % _NBUF]
        )

    gc = [None] * _CHUNKS
    wc = [None] * _CHUNKS
    for g in range(_NBUF):
        gc[g] = gather(g)
    for g in range(_CHUNKS):
        gc[g].wait()
        wc[g] = write(g)
        if g + _NBUF < _CHUNKS:
            wc[g].wait()
            gc[g + _NBUF] = gather(g + _NBUF)
    for g in range(_CHUNKS - _NBUF, _CHUNKS):
        wc[g].wait()


def _run(table):
    mesh = plsc.VectorSubcoreMesh(
        core_axis_name="c", subcore_axis_name="s", num_cores=_NC, num_subcores=_NS
    )
    k = pl.kernel(
        _gather_body,
        out_type=jax.ShapeDtypeStruct((_BATCH, _NUM_KEEP, _DIM), jnp.float32),
        mesh=mesh,
        scratch_types=(
            [pltpu.VMEM((_CHUNKS, _CHUNK), jnp.int32)]
            + [pltpu.VMEM((_CHUNK, _DIM), jnp.float32)] * _NBUF
            + [pltpu.SemaphoreType.DMA] * (2 * _NBUF)
        ),
    )
    return k(table, _flat_indices())


def kernel(patches):
    table = patches.reshape(_BATCH * _NUM_PATCHES, _DIM)
    return _run(table)
